# Initial kernel scaffold; baseline (speedup 1.0000x reference)
import jax
import jax.numpy as jnp
from jax.experimental import pallas as pl


def kernel(geoms, bonds):
    raise NotImplementedError("write your pallas kernel here")


# speedup vs baseline: 17.3547x; 17.3547x over previous
import functools

import jax
import jax.numpy as jnp
from jax import lax
from jax.experimental import pallas as pl
from jax.experimental.pallas import tpu as pltpu
from jax.experimental.pallas import tpu_sc as plsc

NC = 2
NS = 16
NW = NC * NS
CHUNK = 128
NBUF = 2


def _dist_chunk(buf_a, buf_b, out_b, ncoord, nconf):
  nhalf = nconf // 16

  @plsc.parallel_loop(0, CHUNK, 1, unroll=4)
  def _(e):
    for h in range(nhalf):
      acc = None
      for k in range(ncoord):
        a = buf_a[e, pl.ds(k * nconf + h * 16, 16)]
        b = buf_b[e, pl.ds(k * nconf + h * 16, 16)]
        d = a - b
        acc = d * d if acc is None else acc + d * d
      half = acc * 0.5
      i = lax.bitcast_convert_type(acc, jnp.int32)
      i = jnp.int32(0x5F3759DF) - (i >> 1)
      r = lax.bitcast_convert_type(i, jnp.float32)
      r = r * (1.5 - (half * r) * r)
      r = r * (1.5 - (half * r) * r)
      out_b[e, pl.ds(h * 16, 16)] = acc * r


@functools.partial(jax.jit, static_argnames=("n_chunks", "ncoord", "nconf"))
def _sc_distances(table, idx_a, idx_b, *, n_chunks, ncoord, nconf):
  d = ncoord * nconf
  e_pad = NW * n_chunks * CHUNK
  mesh = plsc.VectorSubcoreMesh(core_axis_name="c", subcore_axis_name="s")

  @functools.partial(
      pl.kernel,
      out_type=jax.ShapeDtypeStruct((e_pad, nconf), jnp.float32),
      mesh=mesh,
      compiler_params=pltpu.CompilerParams(use_tc_tiling_on_sc=False),
      scratch_types=[
          pltpu.VMEM((n_chunks, CHUNK), jnp.int32),
          pltpu.VMEM((n_chunks, CHUNK), jnp.int32),
          pltpu.VMEM((NBUF, CHUNK, d), jnp.float32),
          pltpu.VMEM((NBUF, CHUNK, d), jnp.float32),
          pltpu.VMEM((NBUF, CHUNK, nconf), jnp.float32),
          [pltpu.SemaphoreType.DMA] * NBUF,
          [pltpu.SemaphoreType.DMA] * NBUF,
          [pltpu.SemaphoreType.DMA] * NBUF,
      ],
  )
  def run(table_h, idxa_h, idxb_h, out_h, idxa_v, idxb_v, buf_a, buf_b,
          out_v, sem_a, sem_b, sem_o):
    wid = lax.axis_index("s") * NC + lax.axis_index("c")
    pltpu.sync_copy(idxa_h.at[wid], idxa_v)
    pltpu.sync_copy(idxb_h.at[wid], idxb_v)

    def fire_gather(j, b):
      pltpu.async_copy(table_h.at[idxa_v.at[j]], buf_a.at[b], sem_a[b])
      pltpu.async_copy(table_h.at[idxb_v.at[j]], buf_b.at[b], sem_b[b])

    def wait_gather(j, b):
      pltpu.make_async_copy(
          table_h.at[idxa_v.at[j]], buf_a.at[b], sem_a[b]).wait()
      pltpu.make_async_copy(
          table_h.at[idxb_v.at[j]], buf_b.at[b], sem_b[b]).wait()

    def out_slab(j):
      return out_h.at[pl.ds((wid * n_chunks + j) * CHUNK, CHUNK)]

    def fire_out(j, b):
      pltpu.async_copy(out_v.at[b], out_slab(j), sem_o[b])

    def wait_out(j, b):
      pltpu.make_async_copy(out_v.at[b], out_slab(j), sem_o[b]).wait()

    for b in range(NBUF):
      fire_gather(b, b)

    @pl.loop(0, n_chunks, step=NBUF)
    def _(j0):
      for b in range(NBUF):
        j = j0 + b
        wait_gather(j, b)

        @pl.when(j >= NBUF)
        def _():
          wait_out(j, b)

        _dist_chunk(buf_a.at[b], buf_b.at[b], out_v.at[b], ncoord, nconf)
        fire_out(j, b)

        @pl.when(j + NBUF < n_chunks)
        def _():
          fire_gather(j + NBUF, b)

    for b in range(NBUF):
      wait_out(n_chunks - NBUF + b, b)

  return run(table, idx_a, idx_b)


def kernel(geoms, bonds):
  n_atoms, ncoord, nconf = geoms.shape
  table = geoms.reshape(n_atoms, ncoord * nconf)
  n_edges = bonds.shape[0]
  bonds = bonds.astype(jnp.int32)

  slab = NW * CHUNK
  n_chunks = -(-n_edges // slab)
  n_chunks += (-n_chunks) % NBUF
  e_pad = n_chunks * slab
  idx = jnp.pad(bonds, ((0, e_pad - n_edges), (0, 0)))
  idx_a = idx[:, 0].reshape(NW, n_chunks, CHUNK)
  idx_b = idx[:, 1].reshape(NW, n_chunks, CHUNK)

  out = _sc_distances(
      table, idx_a, idx_b, n_chunks=n_chunks, ncoord=ncoord, nconf=nconf)
  return out[:n_edges]


# speedup vs baseline: 22.8925x; 1.3191x over previous
import functools

import jax
import jax.numpy as jnp
from jax import lax
from jax.experimental import pallas as pl
from jax.experimental.pallas import tpu as pltpu
from jax.experimental.pallas import tpu_sc as plsc

NC = 2
NS = 16
NW = NC * NS
CHUNK = 128
NBUF = 2


def _dist_chunk(buf_a, buf_b, out_b, ncoord, nconf):
  nhalf = nconf // 16

  @plsc.parallel_loop(0, CHUNK, 1, unroll=4)
  def _(e):
    for h in range(nhalf):
      acc = None
      for k in range(ncoord):
        a = buf_a[e, pl.ds(k * nconf + h * 16, 16)]
        b = buf_b[e, pl.ds(k * nconf + h * 16, 16)]
        d = a - b
        acc = d * d if acc is None else acc + d * d
      half = acc * 0.5
      i = lax.bitcast_convert_type(acc, jnp.int32)
      i = jnp.int32(0x5F3759DF) - (i >> 1)
      r = lax.bitcast_convert_type(i, jnp.float32)
      r = r * (1.5 - (half * r) * r)
      out_b[e, pl.ds(h * 16, 16)] = acc * r


@functools.partial(
    jax.jit, static_argnames=("n_edges", "n_chunks", "ncoord", "nconf"))
def _sc_distances(table, bonds, *, n_edges, n_chunks, ncoord, nconf):
  d = ncoord * nconf
  mesh = plsc.VectorSubcoreMesh(core_axis_name="c", subcore_axis_name="s")
  last_start = n_edges - CHUNK

  @functools.partial(
      pl.kernel,
      out_type=jax.ShapeDtypeStruct((n_edges, nconf), jnp.float32),
      mesh=mesh,
      compiler_params=pltpu.CompilerParams(
          use_tc_tiling_on_sc=False, needs_layout_passes=False),
      scratch_types=[
          pltpu.VMEM((NBUF, CHUNK, 2), jnp.int32),
          pltpu.VMEM((NBUF * 2, CHUNK), jnp.int32),
          pltpu.VMEM((NBUF, CHUNK, d), jnp.float32),
          pltpu.VMEM((NBUF, CHUNK, d), jnp.float32),
          pltpu.VMEM((NBUF, CHUNK, nconf), jnp.float32),
          [pltpu.SemaphoreType.DMA] * NBUF,
          [pltpu.SemaphoreType.DMA] * NBUF,
          [pltpu.SemaphoreType.DMA] * NBUF,
          [pltpu.SemaphoreType.DMA] * NBUF,
      ],
  )
  def run(table_h, bonds_h, out_h, bond_v, idx_st, buf_a, buf_b, out_v,
          sem_p, sem_a, sem_b, sem_o):
    wid = lax.axis_index("s") * NC + lax.axis_index("c")

    def start(j):
      return jnp.minimum((wid * n_chunks + j) * CHUNK, last_start)

    def fire_bonds(j, s):
      pltpu.async_copy(
          bonds_h.at[pl.ds(start(j), CHUNK)], bond_v.at[s], sem_p[s])

    def wait_bonds(j, s):
      pltpu.make_async_copy(
          bonds_h.at[pl.ds(start(j), CHUNK)], bond_v.at[s], sem_p[s]).wait()

    def deint(s):
      bv = bond_v.at[s]
      lanes = lax.iota(jnp.int32, 16)
      col0 = jnp.zeros((16,), jnp.int32)
      col1 = jnp.ones((16,), jnp.int32)
      for half in range(CHUNK // 16):
        rows = half * 16 + lanes
        idx_st[2 * s, pl.ds(half * 16, 16)] = plsc.load_gather(
            bv, [rows, col0])
        idx_st[2 * s + 1, pl.ds(half * 16, 16)] = plsc.load_gather(
            bv, [rows, col1])

    def fire_gather(s):
      pltpu.async_copy(
          table_h.at[idx_st.at[2 * s]], buf_a.at[s], sem_a[s])
      pltpu.async_copy(
          table_h.at[idx_st.at[2 * s + 1]], buf_b.at[s], sem_b[s])

    def wait_gather(s):
      pltpu.make_async_copy(
          table_h.at[idx_st.at[2 * s]], buf_a.at[s], sem_a[s]).wait()
      pltpu.make_async_copy(
          table_h.at[idx_st.at[2 * s + 1]], buf_b.at[s], sem_b[s]).wait()

    def fire_out(j, s):
      pltpu.async_copy(
          out_v.at[s], out_h.at[pl.ds(start(j), CHUNK)], sem_o[s])

    def wait_out(j, s):
      pltpu.make_async_copy(
          out_v.at[s], out_h.at[pl.ds(start(j), CHUNK)], sem_o[s]).wait()

    fire_bonds(0, 0)
    fire_bonds(1, 1)
    wait_bonds(0, 0)
    deint(0)
    fire_gather(0)

    @pl.loop(0, n_chunks, step=NBUF)
    def _(j0):
      for b in range(NBUF):
        j = j0 + b
        nxt = 1 - b

        @pl.when(j + 1 < n_chunks)
        def _():
          wait_bonds(j + 1, nxt)
          deint(nxt)
          fire_gather(nxt)

        @pl.when(j + 2 < n_chunks)
        def _():
          fire_bonds(j + 2, b)

        wait_gather(b)

        @pl.when(j >= NBUF)
        def _():
          wait_out(j, b)

        _dist_chunk(buf_a.at[b], buf_b.at[b], out_v.at[b], ncoord, nconf)
        fire_out(j, b)

    for b in range(NBUF):
      wait_out(n_chunks - NBUF + b, b)

  return run(table, bonds)


def kernel(geoms, bonds):
  n_atoms, ncoord, nconf = geoms.shape
  table = geoms.reshape(n_atoms, ncoord * nconf)
  n_edges = bonds.shape[0]
  bonds = bonds.astype(jnp.int32)

  n_chunks = -(-n_edges // (NW * CHUNK))
  n_chunks += (-n_chunks) % NBUF

  return _sc_distances(
      table, bonds, n_edges=n_edges, n_chunks=n_chunks, ncoord=ncoord,
      nconf=nconf)
